# Initial kernel scaffold; baseline (speedup 1.0000x reference)
import functools

import jax
import jax.numpy as jnp
from jax import lax
from jax.experimental import pallas as pl
from jax.experimental.pallas import tpu as pltpu
from jax.experimental.pallas import tpu_sc as plsc

N = 50000
E = 800000
IN = 128
H = 4
F = 16
HF = H * F

NC = 2
NS = 16
NW = NC * NS

CHUNK = 128
E_PER_W = 25600
EPAD = E_PER_W * NW
NCHUNK = E_PER_W // CHUNK
RPS = 3128
NPAD = RPS * NS

_mesh = plsc.VectorSubcoreMesh(core_axis_name="c", subcore_axis_name="s")


def _iota16():
  return lax.iota(jnp.int32, 16)


def _col(v):
  return jnp.full((16,), v, jnp.int32)


BN = 2500


def _proj_body(x_ref, wv_ref, bv_ref, wqk_ref, bqk_ref, h_ref, qk_ref,
               qkmax_ref):
  xb = x_ref[...]
  h = jnp.dot(xb, wv_ref[...], preferred_element_type=jnp.float32) + bv_ref[...]
  qk = jnp.dot(h, wqk_ref[...], preferred_element_type=jnp.float32) + bqk_ref[...]
  h_ref[...] = h
  qk_ref[...] = qk
  bm = jnp.max(qk, axis=0, keepdims=True)

  @pl.when(pl.program_id(0) == 0)
  def _():
    qkmax_ref[...] = bm

  @pl.when(pl.program_id(0) > 0)
  def _():
    qkmax_ref[...] = jnp.maximum(qkmax_ref[...], bm)


def _proj(x, wv, bv2, wqk, bqk2):
  return pl.pallas_call(
      _proj_body,
      grid=(N // BN,),
      in_specs=[
          pl.BlockSpec((BN, IN), lambda i: (i, 0)),
          pl.BlockSpec((IN, HF), lambda i: (0, 0)),
          pl.BlockSpec((1, HF), lambda i: (0, 0)),
          pl.BlockSpec((HF, 16), lambda i: (0, 0)),
          pl.BlockSpec((1, 16), lambda i: (0, 0)),
      ],
      out_specs=[
          pl.BlockSpec((BN, HF), lambda i: (i, 0)),
          pl.BlockSpec((BN, 16), lambda i: (i, 0)),
          pl.BlockSpec((1, 16), lambda i: (0, 0)),
      ],
      out_shape=[
          jax.ShapeDtypeStruct((N, HF), jnp.float32),
          jax.ShapeDtypeStruct((N, 16), jnp.float32),
          jax.ShapeDtypeStruct((1, 16), jnp.float32),
      ],
  )(x, wv, bv2, wqk, bqk2)


def _pass1_body(qk, srcp, dstp, shift, z4, ex_out, dparts,
                shiftb, srcb, dstb, qs, kd, exb, denom_sh):
  c = lax.axis_index("c")
  s = lax.axis_index("s")
  wid = c * NS + s

  pltpu.sync_copy(shift.at[pl.ds(0, 16)], shiftb)
  pltpu.sync_copy(z4.at[pl.ds(s * RPS, RPS)], denom_sh.at[pl.ds(s * RPS, RPS)])
  plsc.subcore_barrier()

  sh0 = shiftb[0]
  sh1 = shiftb[1]
  sh2 = shiftb[2]
  sh3 = shiftb[3]
  ebase = wid * E_PER_W

  def chunk_body(i, carry):
    base = ebase + i * CHUNK
    pltpu.sync_copy(srcp.at[pl.ds(base, CHUNK)], srcb)
    pltpu.sync_copy(dstp.at[pl.ds(base, CHUNK)], dstb)
    pltpu.sync_copy(qk.at[srcb], qs)
    pltpu.sync_copy(qk.at[dstb], kd)
    for j in range(CHUNK // 16):
      rows = _col(j * 16) + _iota16()
      gid = base + j * 16 + _iota16()
      valid = gid < E
      for h, sh in ((0, sh0), (1, sh1), (2, sh2), (3, sh3)):
        qv = plsc.load_gather(qs, [rows, _col(h)])
        kv = plsc.load_gather(kd, [rows, _col(4 + h)])
        e = qv + kv
        cf = jnp.where(e >= 0.0, e, 0.2 * e)
        ex = jnp.exp(cf - sh)
        ex = jnp.where(valid, ex, 0.0)
        plsc.store_scatter(exb, [rows, _col(h)], ex)
    pltpu.sync_copy(exb, ex_out.at[pl.ds(base, CHUNK)])
    pltpu.sync_copy(exb, denom_sh.at[dstb], add=True)
    return carry

  lax.fori_loop(0, NCHUNK, chunk_body, 0)
  plsc.subcore_barrier()
  pltpu.sync_copy(denom_sh.at[pl.ds(s * RPS, RPS)],
                  dparts.at[c, pl.ds(s * RPS, RPS)])


def _pass1(qk, srcp, dstp, shift, z4):
  return pl.kernel(
      _pass1_body,
      out_type=[
          jax.ShapeDtypeStruct((EPAD, 4), jnp.float32),
          jax.ShapeDtypeStruct((NC, NPAD, 4), jnp.float32),
      ],
      mesh=_mesh,
      scratch_types=[
          pltpu.VMEM((16,), jnp.float32),
          pltpu.VMEM((CHUNK,), jnp.int32),
          pltpu.VMEM((CHUNK,), jnp.int32),
          pltpu.VMEM((CHUNK, 16), jnp.float32),
          pltpu.VMEM((CHUNK, 16), jnp.float32),
          pltpu.VMEM((CHUNK, 4), jnp.float32),
          pltpu.VMEM_SHARED((NPAD, 4), jnp.float32),
      ],
  )(qk, srcp, dstp, shift, z4)


def _invd_body(a_ref, b_ref, o_ref):
  o_ref[...] = 1.0 / (a_ref[...] + b_ref[...] + 1e-9)


def _invd(da, db):
  return pl.pallas_call(
      _invd_body,
      grid=(NS,),
      in_specs=[
          pl.BlockSpec((RPS, 4), lambda i: (i, 0)),
          pl.BlockSpec((RPS, 4), lambda i: (i, 0)),
      ],
      out_specs=pl.BlockSpec((RPS, 4), lambda i: (i, 0)),
      out_shape=jax.ShapeDtypeStruct((NPAD, 4), jnp.float32),
  )(da, db)


def _pass2_body(hm, ex, invd, srcp, dstp, z16, vparts,
                srcb, dstb, exb, ivb, hsb, msgb, vsh):
  c = lax.axis_index("c")
  s = lax.axis_index("s")
  wid = c * NS + s

  pltpu.sync_copy(z16.at[pl.ds(s * RPS, RPS)], vsh.at[pl.ds(s * RPS, RPS)])
  plsc.subcore_barrier()

  ebase = wid * E_PER_W

  def chunk_body(i, carry):
    base = ebase + i * CHUNK
    pltpu.sync_copy(srcp.at[pl.ds(base, CHUNK)], srcb)
    pltpu.sync_copy(dstp.at[pl.ds(base, CHUNK)], dstb)
    pltpu.sync_copy(ex.at[pl.ds(base, CHUNK)], exb)
    pltpu.sync_copy(invd.at[dstb], ivb)
    pltpu.sync_copy(hm.at[srcb], hsb)
    for j in range(CHUNK // 16):
      rows = _col(j * 16) + _iota16()
      a0 = plsc.load_gather(exb, [rows, _col(0)]) * plsc.load_gather(
          ivb, [rows, _col(0)]) * 0.25
      a1 = plsc.load_gather(exb, [rows, _col(1)]) * plsc.load_gather(
          ivb, [rows, _col(1)]) * 0.25
      a2 = plsc.load_gather(exb, [rows, _col(2)]) * plsc.load_gather(
          ivb, [rows, _col(2)]) * 0.25
      a3 = plsc.load_gather(exb, [rows, _col(3)]) * plsc.load_gather(
          ivb, [rows, _col(3)]) * 0.25
      for f in range(F):
        m = a0 * plsc.load_gather(hsb, [rows, _col(f)])
        m = m + a1 * plsc.load_gather(hsb, [rows, _col(16 + f)])
        m = m + a2 * plsc.load_gather(hsb, [rows, _col(32 + f)])
        m = m + a3 * plsc.load_gather(hsb, [rows, _col(48 + f)])
        plsc.store_scatter(msgb, [rows, _col(f)], m)
    pltpu.sync_copy(msgb, vsh.at[dstb], add=True)
    return carry

  lax.fori_loop(0, NCHUNK, chunk_body, 0)
  plsc.subcore_barrier()
  pltpu.sync_copy(vsh.at[pl.ds(s * RPS, RPS)],
                  vparts.at[c, pl.ds(s * RPS, RPS)])


def _pass2(hm, ex, invd, srcp, dstp, z16):
  return pl.kernel(
      _pass2_body,
      out_type=jax.ShapeDtypeStruct((NC, NPAD, 16), jnp.float32),
      mesh=_mesh,
      scratch_types=[
          pltpu.VMEM((CHUNK,), jnp.int32),
          pltpu.VMEM((CHUNK,), jnp.int32),
          pltpu.VMEM((CHUNK, 4), jnp.float32),
          pltpu.VMEM((CHUNK, 4), jnp.float32),
          pltpu.VMEM((CHUNK, HF), jnp.float32),
          pltpu.VMEM((CHUNK, 16), jnp.float32),
          pltpu.VMEM_SHARED((NPAD, 16), jnp.float32),
      ],
  )(hm, ex, invd, srcp, dstp, z16)


def _comb_body(a_ref, b_ref, o_ref):
  o_ref[...] = a_ref[...] + b_ref[...]


def _combine(va, vb):
  return pl.pallas_call(
      _comb_body,
      grid=(N // BN,),
      in_specs=[
          pl.BlockSpec((BN, 16), lambda i: (i, 0)),
          pl.BlockSpec((BN, 16), lambda i: (i, 0)),
      ],
      out_specs=pl.BlockSpec((BN, 16), lambda i: (i, 0)),
      out_shape=jax.ShapeDtypeStruct((N, 16), jnp.float32),
  )(va, vb)


@jax.jit
def kernel(x, edge_index, W_v, b_v, W_q, b_q, W_k, b_k):
  wqk = jnp.concatenate(
      [W_q, W_k, jnp.zeros((HF, 16 - 2 * H), jnp.float32)], axis=1)
  bqk2 = jnp.concatenate(
      [b_q, b_k, jnp.zeros((16 - 2 * H,), jnp.float32)]).reshape(1, 16)
  bv2 = b_v.reshape(1, HF)

  hm, qk, qkmax = _proj(x, W_v, bv2, wqk, bqk2)
  qm = qkmax[0]
  shift4 = jnp.maximum(qm[:H] + qm[H:2 * H], 0.0)
  shift = jnp.concatenate([shift4, jnp.zeros((12,), jnp.float32)])

  pad = jnp.zeros((EPAD - E,), jnp.int32)
  srcp = jnp.concatenate([edge_index[0], pad])
  dstp = jnp.concatenate([edge_index[1], pad])

  z4 = jnp.zeros((NPAD, 4), jnp.float32)
  z16 = jnp.zeros((NPAD, 16), jnp.float32)

  ex, dparts = _pass1(qk, srcp, dstp, shift, z4)
  invd = _invd(dparts[0], dparts[1])
  vparts = _pass2(hm, ex, invd, srcp, dstp, z16)
  return _combine(vparts[0], vparts[1])


# speedup vs baseline: 30.2495x; 30.2495x over previous
import functools

import jax
import jax.numpy as jnp
from jax import lax
from jax.experimental import pallas as pl
from jax.experimental.pallas import tpu as pltpu
from jax.experimental.pallas import tpu_sc as plsc

N = 50000
E = 800000
IN = 128
H = 4
F = 16
HF = H * F

NC = 2
NS = 16
NW = NC * NS

CHUNK = 128
E_PER_W = 25600
EPAD = E_PER_W * NW
NCHUNK = E_PER_W // CHUNK
RPS = 3128
NPAD = RPS * NS

_mesh = plsc.VectorSubcoreMesh(core_axis_name="c", subcore_axis_name="s")


def _iota16():
  return lax.iota(jnp.int32, 16)


def _col(v):
  return jnp.full((16,), v, jnp.int32)


BN = 2000


def _proj_body(x_ref, wv_ref, bv_ref, wqk_ref, bqk_ref, h_ref, qk_ref,
               qkmax_ref):
  xb = x_ref[...]
  h = jnp.dot(xb, wv_ref[...], preferred_element_type=jnp.float32) + bv_ref[...]
  qk = jnp.dot(h, wqk_ref[...], preferred_element_type=jnp.float32) + bqk_ref[...]
  h_ref[...] = h
  qk_ref[...] = qk
  bm = jnp.max(qk, axis=0, keepdims=True)

  @pl.when(pl.program_id(0) == 0)
  def _():
    qkmax_ref[...] = bm

  @pl.when(pl.program_id(0) > 0)
  def _():
    qkmax_ref[...] = jnp.maximum(qkmax_ref[...], bm)


def _proj(x, wv, bv2, wqk, bqk2):
  return pl.pallas_call(
      _proj_body,
      grid=(N // BN,),
      in_specs=[
          pl.BlockSpec((BN, IN), lambda i: (i, 0)),
          pl.BlockSpec((IN, HF), lambda i: (0, 0)),
          pl.BlockSpec((1, HF), lambda i: (0, 0)),
          pl.BlockSpec((HF, 16), lambda i: (0, 0)),
          pl.BlockSpec((1, 16), lambda i: (0, 0)),
      ],
      out_specs=[
          pl.BlockSpec((BN, HF), lambda i: (i, 0)),
          pl.BlockSpec((BN, 16), lambda i: (i, 0)),
          pl.BlockSpec((1, 16), lambda i: (0, 0)),
      ],
      out_shape=[
          jax.ShapeDtypeStruct((N, HF), jnp.float32),
          jax.ShapeDtypeStruct((N, 16), jnp.float32),
          jax.ShapeDtypeStruct((1, 16), jnp.float32),
      ],
  )(x, wv, bv2, wqk, bqk2)


def _pass1_body(qk, srcp, dstp, shift, z16, ex_out, dparts,
                shiftb, srcb, dstb, qs, kd, exb, exb16, denom_sh):
  c = lax.axis_index("c")
  s = lax.axis_index("s")
  wid = c * NS + s

  pltpu.sync_copy(shift.at[pl.ds(0, 16)], shiftb)
  pltpu.sync_copy(z16.at[pl.ds(s * RPS, RPS)], denom_sh.at[pl.ds(s * RPS, RPS)])
  for r in range(CHUNK):
    exb16[r, :] = jnp.zeros((16,), jnp.float32)
  plsc.subcore_barrier()

  shv = shiftb[...]
  sh0 = shv[0]
  sh1 = shv[1]
  sh2 = shv[2]
  sh3 = shv[3]
  ebase = wid * E_PER_W

  def chunk_body(i, carry):
    base = ebase + i * CHUNK
    pltpu.sync_copy(srcp.at[pl.ds(base, CHUNK)], srcb)
    pltpu.sync_copy(dstp.at[pl.ds(base, CHUNK)], dstb)
    pltpu.sync_copy(qk.at[srcb], qs)
    pltpu.sync_copy(qk.at[dstb], kd)
    for j in range(CHUNK // 16):
      rows = _col(j * 16) + _iota16()
      gid = base + j * 16 + _iota16()
      valid = gid < E
      for h, sh in ((0, sh0), (1, sh1), (2, sh2), (3, sh3)):
        qv = plsc.load_gather(qs, [rows, _col(h)])
        kv = plsc.load_gather(kd, [rows, _col(4 + h)])
        e = qv + kv
        cf = jnp.where(e >= 0.0, e, 0.2 * e)
        ex = jnp.exp(cf - sh)
        ex = jnp.where(valid, ex, 0.0)
        plsc.store_scatter(exb, [rows, _col(h)], ex)
        plsc.store_scatter(exb16, [rows, _col(h)], ex)
    pltpu.sync_copy(exb, ex_out.at[pl.ds(base, CHUNK)])
    pltpu.sync_copy(exb16, denom_sh.at[dstb], add=True)
    return carry

  lax.fori_loop(0, NCHUNK, chunk_body, 0)
  plsc.subcore_barrier()
  pltpu.sync_copy(denom_sh.at[pl.ds(s * RPS, RPS)],
                  dparts.at[c, pl.ds(s * RPS, RPS)])


def _pass1(qk, srcp, dstp, shift, z16):
  return pl.kernel(
      _pass1_body,
      out_type=[
          jax.ShapeDtypeStruct((EPAD, 4), jnp.float32),
          jax.ShapeDtypeStruct((NC, NPAD, 16), jnp.float32),
      ],
      mesh=_mesh,
      scratch_types=[
          pltpu.VMEM((16,), jnp.float32),
          pltpu.VMEM((CHUNK,), jnp.int32),
          pltpu.VMEM((CHUNK,), jnp.int32),
          pltpu.VMEM((CHUNK, 16), jnp.float32),
          pltpu.VMEM((CHUNK, 16), jnp.float32),
          pltpu.VMEM((CHUNK, 4), jnp.float32),
          pltpu.VMEM((CHUNK, 16), jnp.float32),
          pltpu.VMEM_SHARED((NPAD, 16), jnp.float32),
      ],
      compiler_params=pltpu.CompilerParams(needs_layout_passes=False, use_tc_tiling_on_sc=False),
  )(qk, srcp, dstp, shift, z16)


def _invd_body(a_ref, b_ref, o_ref):
  o_ref[...] = 1.0 / (a_ref[...] + b_ref[...] + 1e-9)


def _invd(da, db):
  return pl.pallas_call(
      _invd_body,
      grid=(NS,),
      in_specs=[
          pl.BlockSpec((RPS, 16), lambda i: (i, 0)),
          pl.BlockSpec((RPS, 16), lambda i: (i, 0)),
      ],
      out_specs=pl.BlockSpec((RPS, 16), lambda i: (i, 0)),
      out_shape=jax.ShapeDtypeStruct((NPAD, 16), jnp.float32),
  )(da, db)


def _pass2_body(hm, ex, invd, srcp, dstp, z16, vparts,
                srcb, dstb, exb, ivb, hsb, msgb, vsh):
  c = lax.axis_index("c")
  s = lax.axis_index("s")
  wid = c * NS + s

  pltpu.sync_copy(z16.at[pl.ds(s * RPS, RPS)], vsh.at[pl.ds(s * RPS, RPS)])
  plsc.subcore_barrier()

  ebase = wid * E_PER_W

  def chunk_body(i, carry):
    base = ebase + i * CHUNK
    pltpu.sync_copy(srcp.at[pl.ds(base, CHUNK)], srcb)
    pltpu.sync_copy(dstp.at[pl.ds(base, CHUNK)], dstb)
    pltpu.sync_copy(ex.at[pl.ds(base, CHUNK)], exb)
    pltpu.sync_copy(invd.at[dstb], ivb)
    pltpu.sync_copy(hm.at[srcb], hsb)
    for j in range(CHUNK // 16):
      rows = _col(j * 16) + _iota16()
      a0 = plsc.load_gather(exb, [rows, _col(0)]) * plsc.load_gather(
          ivb, [rows, _col(0)]) * 0.25
      a1 = plsc.load_gather(exb, [rows, _col(1)]) * plsc.load_gather(
          ivb, [rows, _col(1)]) * 0.25
      a2 = plsc.load_gather(exb, [rows, _col(2)]) * plsc.load_gather(
          ivb, [rows, _col(2)]) * 0.25
      a3 = plsc.load_gather(exb, [rows, _col(3)]) * plsc.load_gather(
          ivb, [rows, _col(3)]) * 0.25
      for f in range(F):
        m = a0 * plsc.load_gather(hsb, [rows, _col(f)])
        m = m + a1 * plsc.load_gather(hsb, [rows, _col(16 + f)])
        m = m + a2 * plsc.load_gather(hsb, [rows, _col(32 + f)])
        m = m + a3 * plsc.load_gather(hsb, [rows, _col(48 + f)])
        plsc.store_scatter(msgb, [rows, _col(f)], m)
    pltpu.sync_copy(msgb, vsh.at[dstb], add=True)
    return carry

  lax.fori_loop(0, NCHUNK, chunk_body, 0)
  plsc.subcore_barrier()
  pltpu.sync_copy(vsh.at[pl.ds(s * RPS, RPS)],
                  vparts.at[c, pl.ds(s * RPS, RPS)])


def _pass2(hm, ex, invd, srcp, dstp, z16):
  return pl.kernel(
      _pass2_body,
      out_type=jax.ShapeDtypeStruct((NC, NPAD, 16), jnp.float32),
      mesh=_mesh,
      scratch_types=[
          pltpu.VMEM((CHUNK,), jnp.int32),
          pltpu.VMEM((CHUNK,), jnp.int32),
          pltpu.VMEM((CHUNK, 4), jnp.float32),
          pltpu.VMEM((CHUNK, 16), jnp.float32),
          pltpu.VMEM((CHUNK, HF), jnp.float32),
          pltpu.VMEM((CHUNK, 16), jnp.float32),
          pltpu.VMEM_SHARED((NPAD, 16), jnp.float32),
      ],
      compiler_params=pltpu.CompilerParams(needs_layout_passes=False, use_tc_tiling_on_sc=False),
  )(hm, ex, invd, srcp, dstp, z16)


def _comb_body(a_ref, b_ref, o_ref):
  o_ref[...] = a_ref[...] + b_ref[...]


def _combine(va, vb):
  return pl.pallas_call(
      _comb_body,
      grid=(N // BN,),
      in_specs=[
          pl.BlockSpec((BN, 16), lambda i: (i, 0)),
          pl.BlockSpec((BN, 16), lambda i: (i, 0)),
      ],
      out_specs=pl.BlockSpec((BN, 16), lambda i: (i, 0)),
      out_shape=jax.ShapeDtypeStruct((N, 16), jnp.float32),
  )(va, vb)


@jax.jit
def kernel(x, edge_index, W_v, b_v, W_q, b_q, W_k, b_k):
  wqk = jnp.concatenate(
      [W_q, W_k, jnp.zeros((HF, 16 - 2 * H), jnp.float32)], axis=1)
  bqk2 = jnp.concatenate(
      [b_q, b_k, jnp.zeros((16 - 2 * H,), jnp.float32)]).reshape(1, 16)
  bv2 = b_v.reshape(1, HF)

  hm, qk, qkmax = _proj(x, W_v, bv2, wqk, bqk2)
  qm = qkmax[0]
  shift4 = jnp.maximum(qm[:H] + qm[H:2 * H], 0.0)
  shift = jnp.concatenate([shift4, jnp.zeros((12,), jnp.float32)])

  pad = jnp.zeros((EPAD - E,), jnp.int32)
  srcp = jnp.concatenate([edge_index[0], pad])
  dstp = jnp.concatenate([edge_index[1], pad])

  z16 = jnp.zeros((NPAD, 16), jnp.float32)

  ex, dparts = _pass1(qk, srcp, dstp, shift, z16)
  invd = _invd(dparts[0], dparts[1])
  vparts = _pass2(hm, ex, invd, srcp, dstp, z16)
  return _combine(vparts[0], vparts[1])


# speedup vs baseline: 37.1212x; 1.2272x over previous
import functools

import jax
import jax.numpy as jnp
from jax import lax
from jax.experimental import pallas as pl
from jax.experimental.pallas import tpu as pltpu
from jax.experimental.pallas import tpu_sc as plsc

N = 50000
E = 800000
IN = 128
H = 4
F = 16
HF = H * F

NC = 2
NS = 16
NW = NC * NS

CHUNK = 128
E_PER_W = 25600
EPAD = E_PER_W * NW
NCHUNK = E_PER_W // CHUNK
RPS = 3128
NPAD = RPS * NS

_mesh = plsc.VectorSubcoreMesh(core_axis_name="c", subcore_axis_name="s")


def _iota16():
  return lax.iota(jnp.int32, 16)


def _col(v):
  return jnp.full((16,), v, jnp.int32)


BN = 2000


def _proj_body(x_ref, wv_ref, bv_ref, wqk_ref, bqk_ref, h_ref, qk_ref,
               qkmax_ref):
  xb = x_ref[...]
  h = jnp.dot(xb, wv_ref[...], preferred_element_type=jnp.float32) + bv_ref[...]
  qk = jnp.dot(h, wqk_ref[...], preferred_element_type=jnp.float32) + bqk_ref[...]
  h_ref[...] = h
  qk_ref[...] = qk
  bm = jnp.max(qk, axis=0, keepdims=True)

  @pl.when(pl.program_id(0) == 0)
  def _():
    qkmax_ref[...] = bm

  @pl.when(pl.program_id(0) > 0)
  def _():
    qkmax_ref[...] = jnp.maximum(qkmax_ref[...], bm)


def _proj(x, wv, bv2, wqk, bqk2):
  return pl.pallas_call(
      _proj_body,
      grid=(N // BN,),
      in_specs=[
          pl.BlockSpec((BN, IN), lambda i: (i, 0)),
          pl.BlockSpec((IN, HF), lambda i: (0, 0)),
          pl.BlockSpec((1, HF), lambda i: (0, 0)),
          pl.BlockSpec((HF, 16), lambda i: (0, 0)),
          pl.BlockSpec((1, 16), lambda i: (0, 0)),
      ],
      out_specs=[
          pl.BlockSpec((BN, HF), lambda i: (i, 0)),
          pl.BlockSpec((BN, 16), lambda i: (i, 0)),
          pl.BlockSpec((1, 16), lambda i: (0, 0)),
      ],
      out_shape=[
          jax.ShapeDtypeStruct((N, HF), jnp.float32),
          jax.ShapeDtypeStruct((N, 16), jnp.float32),
          jax.ShapeDtypeStruct((1, 16), jnp.float32),
      ],
  )(x, wv, bv2, wqk, bqk2)


def _pass1_body(qk, srcp, dstp, shift, z16, ex_out, dparts,
                shiftb, srcb, dstb, qs, kd, exb, exb16, denom_sh, insem):
  c = lax.axis_index("c")
  s = lax.axis_index("s")
  wid = c * NS + s

  pltpu.sync_copy(shift.at[pl.ds(0, 16)], shiftb)
  pltpu.sync_copy(z16.at[pl.ds(s * RPS, RPS)], denom_sh.at[pl.ds(s * RPS, RPS)])
  for r in range(CHUNK):
    exb16[r, :] = jnp.zeros((16,), jnp.float32)
  plsc.subcore_barrier()

  shv = shiftb[...]
  sh0 = shv[0]
  sh1 = shv[1]
  sh2 = shv[2]
  sh3 = shv[3]
  ebase = wid * E_PER_W

  def chunk_body(i, carry):
    base = ebase + i * CHUNK
    c1 = pltpu.async_copy(srcp.at[pl.ds(base, CHUNK)], srcb, insem)
    c2 = pltpu.async_copy(dstp.at[pl.ds(base, CHUNK)], dstb, insem)
    c1.wait()
    c2.wait()
    g1 = pltpu.async_copy(qk.at[srcb], qs, insem)
    g2 = pltpu.async_copy(qk.at[dstb], kd, insem)
    g1.wait()
    g2.wait()
    for j in range(CHUNK // 16):
      rows = _col(j * 16) + _iota16()
      gid = base + j * 16 + _iota16()
      valid = gid < E
      for h, sh in ((0, sh0), (1, sh1), (2, sh2), (3, sh3)):
        qv = plsc.load_gather(qs, [rows, _col(h)])
        kv = plsc.load_gather(kd, [rows, _col(4 + h)])
        e = qv + kv
        cf = jnp.where(e >= 0.0, e, 0.2 * e)
        ex = jnp.exp(cf - sh)
        ex = jnp.where(valid, ex, 0.0)
        plsc.store_scatter(exb, [rows, _col(h)], ex)
        plsc.store_scatter(exb16, [rows, _col(h)], ex)
    w1 = pltpu.async_copy(exb, ex_out.at[pl.ds(base, CHUNK)], insem)
    pltpu.sync_copy(exb16, denom_sh.at[dstb], add=True)
    w1.wait()
    return carry

  lax.fori_loop(0, NCHUNK, chunk_body, 0)
  plsc.subcore_barrier()
  pltpu.sync_copy(denom_sh.at[pl.ds(s * RPS, RPS)],
                  dparts.at[c, pl.ds(s * RPS, RPS)])


def _pass1(qk, srcp, dstp, shift, z16):
  return pl.kernel(
      _pass1_body,
      out_type=[
          jax.ShapeDtypeStruct((EPAD, 4), jnp.float32),
          jax.ShapeDtypeStruct((NC, NPAD, 16), jnp.float32),
      ],
      mesh=_mesh,
      scratch_types=[
          pltpu.VMEM((16,), jnp.float32),
          pltpu.VMEM((CHUNK,), jnp.int32),
          pltpu.VMEM((CHUNK,), jnp.int32),
          pltpu.VMEM((CHUNK, 16), jnp.float32),
          pltpu.VMEM((CHUNK, 16), jnp.float32),
          pltpu.VMEM((CHUNK, 4), jnp.float32),
          pltpu.VMEM((CHUNK, 16), jnp.float32),
          pltpu.VMEM_SHARED((NPAD, 16), jnp.float32),
          pltpu.SemaphoreType.DMA,
      ],
      compiler_params=pltpu.CompilerParams(needs_layout_passes=False, use_tc_tiling_on_sc=False),
  )(qk, srcp, dstp, shift, z16)


def _invd_body(a_ref, b_ref, o_ref):
  o_ref[...] = 1.0 / (a_ref[...] + b_ref[...] + 1e-9)


def _invd(da, db):
  return pl.pallas_call(
      _invd_body,
      grid=(NS,),
      in_specs=[
          pl.BlockSpec((RPS, 16), lambda i: (i, 0)),
          pl.BlockSpec((RPS, 16), lambda i: (i, 0)),
      ],
      out_specs=pl.BlockSpec((RPS, 16), lambda i: (i, 0)),
      out_shape=jax.ShapeDtypeStruct((NPAD, 16), jnp.float32),
  )(da, db)


def _pass2_body(hm, ex, invd, srcp, dstp, z16, vparts,
                srcb, dstb, exb, ivb, hsb, msgb, vsh, insem):
  c = lax.axis_index("c")
  s = lax.axis_index("s")
  wid = c * NS + s

  pltpu.sync_copy(z16.at[pl.ds(s * RPS, RPS)], vsh.at[pl.ds(s * RPS, RPS)])
  plsc.subcore_barrier()

  ebase = wid * E_PER_W

  def chunk_body(i, carry):
    base = ebase + i * CHUNK
    c1 = pltpu.async_copy(srcp.at[pl.ds(base, CHUNK)], srcb, insem)
    c2 = pltpu.async_copy(dstp.at[pl.ds(base, CHUNK)], dstb, insem)
    c3 = pltpu.async_copy(ex.at[pl.ds(base, CHUNK)], exb, insem)
    c1.wait()
    c2.wait()
    g1 = pltpu.async_copy(invd.at[dstb], ivb, insem)
    g2 = pltpu.async_copy(hm.at[srcb], hsb, insem)
    c3.wait()
    g1.wait()
    g2.wait()
    for j in range(CHUNK // 16):
      rows = _col(j * 16) + _iota16()
      a0 = plsc.load_gather(exb, [rows, _col(0)]) * plsc.load_gather(
          ivb, [rows, _col(0)]) * 0.25
      a1 = plsc.load_gather(exb, [rows, _col(1)]) * plsc.load_gather(
          ivb, [rows, _col(1)]) * 0.25
      a2 = plsc.load_gather(exb, [rows, _col(2)]) * plsc.load_gather(
          ivb, [rows, _col(2)]) * 0.25
      a3 = plsc.load_gather(exb, [rows, _col(3)]) * plsc.load_gather(
          ivb, [rows, _col(3)]) * 0.25
      for f in range(F):
        m = a0 * plsc.load_gather(hsb, [rows, _col(f)])
        m = m + a1 * plsc.load_gather(hsb, [rows, _col(16 + f)])
        m = m + a2 * plsc.load_gather(hsb, [rows, _col(32 + f)])
        m = m + a3 * plsc.load_gather(hsb, [rows, _col(48 + f)])
        plsc.store_scatter(msgb, [rows, _col(f)], m)
    pltpu.sync_copy(msgb, vsh.at[dstb], add=True)
    return carry

  lax.fori_loop(0, NCHUNK, chunk_body, 0)
  plsc.subcore_barrier()
  pltpu.sync_copy(vsh.at[pl.ds(s * RPS, RPS)],
                  vparts.at[c, pl.ds(s * RPS, RPS)])


def _pass2(hm, ex, invd, srcp, dstp, z16):
  return pl.kernel(
      _pass2_body,
      out_type=jax.ShapeDtypeStruct((NC, NPAD, 16), jnp.float32),
      mesh=_mesh,
      scratch_types=[
          pltpu.VMEM((CHUNK,), jnp.int32),
          pltpu.VMEM((CHUNK,), jnp.int32),
          pltpu.VMEM((CHUNK, 4), jnp.float32),
          pltpu.VMEM((CHUNK, 16), jnp.float32),
          pltpu.VMEM((CHUNK, HF), jnp.float32),
          pltpu.VMEM((CHUNK, 16), jnp.float32),
          pltpu.VMEM_SHARED((NPAD, 16), jnp.float32),
          pltpu.SemaphoreType.DMA,
      ],
      compiler_params=pltpu.CompilerParams(needs_layout_passes=False, use_tc_tiling_on_sc=False),
  )(hm, ex, invd, srcp, dstp, z16)


def _comb_body(a_ref, b_ref, o_ref):
  o_ref[...] = a_ref[...] + b_ref[...]


def _combine(va, vb):
  return pl.pallas_call(
      _comb_body,
      grid=(N // BN,),
      in_specs=[
          pl.BlockSpec((BN, 16), lambda i: (i, 0)),
          pl.BlockSpec((BN, 16), lambda i: (i, 0)),
      ],
      out_specs=pl.BlockSpec((BN, 16), lambda i: (i, 0)),
      out_shape=jax.ShapeDtypeStruct((N, 16), jnp.float32),
  )(va, vb)


@jax.jit
def kernel(x, edge_index, W_v, b_v, W_q, b_q, W_k, b_k):
  wqk = jnp.concatenate(
      [W_q, W_k, jnp.zeros((HF, 16 - 2 * H), jnp.float32)], axis=1)
  bqk2 = jnp.concatenate(
      [b_q, b_k, jnp.zeros((16 - 2 * H,), jnp.float32)]).reshape(1, 16)
  bv2 = b_v.reshape(1, HF)

  hm, qk, qkmax = _proj(x, W_v, bv2, wqk, bqk2)
  qm = qkmax[0]
  shift4 = jnp.maximum(qm[:H] + qm[H:2 * H], 0.0)
  shift = jnp.concatenate([shift4, jnp.zeros((12,), jnp.float32)])

  pad = jnp.zeros((EPAD - E,), jnp.int32)
  srcp = jnp.concatenate([edge_index[0], pad])
  dstp = jnp.concatenate([edge_index[1], pad])

  z16 = jnp.zeros((NPAD, 16), jnp.float32)

  ex, dparts = _pass1(qk, srcp, dstp, shift, z16)
  invd = _invd(dparts[0], dparts[1])
  vparts = _pass2(hm, ex, invd, srcp, dstp, z16)
  return _combine(vparts[0], vparts[1])


# speedup vs baseline: 54.8756x; 1.4783x over previous
import functools

import jax
import jax.numpy as jnp
from jax import lax
from jax.experimental import pallas as pl
from jax.experimental.pallas import tpu as pltpu
from jax.experimental.pallas import tpu_sc as plsc

N = 50000
E = 800000
IN = 128
H = 4
F = 16
HF = H * F

NC = 2
NS = 16
NW = NC * NS

CHUNK = 128
E_PER_W = 25600
EPAD = E_PER_W * NW
NCHUNK = E_PER_W // CHUNK
RPS = 3128
NPAD = RPS * NS

_mesh = plsc.VectorSubcoreMesh(core_axis_name="c", subcore_axis_name="s")


def _iota16():
  return lax.iota(jnp.int32, 16)


def _col(v):
  return jnp.full((16,), v, jnp.int32)


BN = 2000


def _proj_body(x_ref, wv_ref, bv_ref, wqk_ref, bqk_ref, h_ref, qk_ref,
               qkmax_ref):
  xb = x_ref[...]
  h = jnp.dot(xb, wv_ref[...], preferred_element_type=jnp.float32) + bv_ref[...]
  qk = jnp.dot(h, wqk_ref[...], preferred_element_type=jnp.float32) + bqk_ref[...]
  h_ref[...] = h
  qk_ref[...] = qk
  bm = jnp.max(qk, axis=0, keepdims=True)

  @pl.when(pl.program_id(0) == 0)
  def _():
    qkmax_ref[...] = bm

  @pl.when(pl.program_id(0) > 0)
  def _():
    qkmax_ref[...] = jnp.maximum(qkmax_ref[...], bm)


def _proj(x, wv, bv2, wqk, bqk2):
  return pl.pallas_call(
      _proj_body,
      grid=(N // BN,),
      in_specs=[
          pl.BlockSpec((BN, IN), lambda i: (i, 0)),
          pl.BlockSpec((IN, HF), lambda i: (0, 0)),
          pl.BlockSpec((1, HF), lambda i: (0, 0)),
          pl.BlockSpec((HF, 16), lambda i: (0, 0)),
          pl.BlockSpec((1, 16), lambda i: (0, 0)),
      ],
      out_specs=[
          pl.BlockSpec((BN, HF), lambda i: (i, 0)),
          pl.BlockSpec((BN, 16), lambda i: (i, 0)),
          pl.BlockSpec((1, 16), lambda i: (0, 0)),
      ],
      out_shape=[
          jax.ShapeDtypeStruct((N, HF), jnp.float32),
          jax.ShapeDtypeStruct((N, 16), jnp.float32),
          jax.ShapeDtypeStruct((1, 16), jnp.float32),
      ],
  )(x, wv, bv2, wqk, bqk2)


def _pass1_body(qk, srcp, dstp, shift, z16, ex_out, dparts,
                shiftb, srcb, dstb, qs, kd, exb, exb16, denom_sh,
                semA, semB, semW):
  c = lax.axis_index("c")
  s = lax.axis_index("s")
  wid = c * NS + s

  pltpu.sync_copy(shift.at[pl.ds(0, 16)], shiftb)
  pltpu.sync_copy(z16.at[pl.ds(s * RPS, RPS)], denom_sh.at[pl.ds(s * RPS, RPS)])
  for r in range(CHUNK):
    exb16[0, r, :] = jnp.zeros((16,), jnp.float32)
    exb16[1, r, :] = jnp.zeros((16,), jnp.float32)
  plsc.subcore_barrier()

  shv = shiftb[...]
  shs = (shv[0], shv[1], shv[2], shv[3])
  ebase = wid * E_PER_W

  def issueA(cc, b):
    base = ebase + cc * CHUNK
    pltpu.async_copy(srcp.at[pl.ds(base, CHUNK)], srcb.at[b], semA.at[b])
    pltpu.async_copy(dstp.at[pl.ds(base, CHUNK)], dstb.at[b], semA.at[b])

  def waitA(b):
    pltpu.make_async_copy(srcp.at[pl.ds(0, CHUNK)], srcb.at[b], semA.at[b]).wait()
    pltpu.make_async_copy(dstp.at[pl.ds(0, CHUNK)], dstb.at[b], semA.at[b]).wait()

  def issueB(b):
    pltpu.async_copy(qk.at[srcb.at[b]], qs.at[b], semB.at[b])
    pltpu.async_copy(qk.at[dstb.at[b]], kd.at[b], semB.at[b])

  def waitB(b):
    pltpu.make_async_copy(qk.at[srcb.at[b]], qs.at[b], semB.at[b]).wait()
    pltpu.make_async_copy(qk.at[dstb.at[b]], kd.at[b], semB.at[b]).wait()

  issueA(0, 0)
  issueA(1, 1)
  waitA(0)
  issueB(0)

  def step(i, carry):
    for b in range(2):
      cc = 2 * i + b
      b2 = 1 - b
      @pl.when(cc + 1 < NCHUNK)
      def _():
        waitA(b2)
        issueB(b2)
      waitB(b)
      base = ebase + cc * CHUNK

      @pl.when(cc >= 2)
      def _():
        pltpu.make_async_copy(exb.at[b], ex_out.at[pl.ds(0, CHUNK)],
                              semW.at[b]).wait()
      for j in range(CHUNK // 16):
        rows = _col(j * 16) + _iota16()
        gid = base + j * 16 + _iota16()
        valid = gid < E
        for h in range(4):
          qv = plsc.load_gather(qs.at[b], [rows, _col(h)])
          kv = plsc.load_gather(kd.at[b], [rows, _col(4 + h)])
          e = qv + kv
          cf = jnp.where(e >= 0.0, e, 0.2 * e)
          ex = jnp.exp(cf - shs[h])
          ex = jnp.where(valid, ex, 0.0)
          plsc.store_scatter(exb.at[b], [rows, _col(h)], ex)
          plsc.store_scatter(exb16.at[b], [rows, _col(h)], ex)
      pltpu.async_copy(exb.at[b], ex_out.at[pl.ds(base, CHUNK)], semW.at[b])
      pltpu.sync_copy(exb16.at[b], denom_sh.at[dstb.at[b]], add=True)

      @pl.when(cc + 2 < NCHUNK)
      def _():
        issueA(cc + 2, b)
    return carry

  lax.fori_loop(0, NCHUNK // 2, step, 0)
  for b in range(2):
    pltpu.make_async_copy(exb.at[b], ex_out.at[pl.ds(0, CHUNK)],
                          semW.at[b]).wait()
  plsc.subcore_barrier()
  pltpu.sync_copy(denom_sh.at[pl.ds(s * RPS, RPS)],
                  dparts.at[c, pl.ds(s * RPS, RPS)])


def _pass1(qk, srcp, dstp, shift, z16):
  return pl.kernel(
      _pass1_body,
      out_type=[
          jax.ShapeDtypeStruct((EPAD, 4), jnp.float32),
          jax.ShapeDtypeStruct((NC, NPAD, 16), jnp.float32),
      ],
      mesh=_mesh,
      scratch_types=[
          pltpu.VMEM((16,), jnp.float32),
          pltpu.VMEM((2, CHUNK), jnp.int32),
          pltpu.VMEM((2, CHUNK), jnp.int32),
          pltpu.VMEM((2, CHUNK, 16), jnp.float32),
          pltpu.VMEM((2, CHUNK, 16), jnp.float32),
          pltpu.VMEM((2, CHUNK, 4), jnp.float32),
          pltpu.VMEM((2, CHUNK, 16), jnp.float32),
          pltpu.VMEM_SHARED((NPAD, 16), jnp.float32),
          pltpu.SemaphoreType.DMA((2,)),
          pltpu.SemaphoreType.DMA((2,)),
          pltpu.SemaphoreType.DMA((2,)),
      ],
      compiler_params=pltpu.CompilerParams(needs_layout_passes=False, use_tc_tiling_on_sc=False),
  )(qk, srcp, dstp, shift, z16)


def _invd_body(a_ref, b_ref, o_ref):
  o_ref[...] = 1.0 / (a_ref[...] + b_ref[...] + 1e-9)


def _invd(da, db):
  return pl.pallas_call(
      _invd_body,
      grid=(NS,),
      in_specs=[
          pl.BlockSpec((RPS, 16), lambda i: (i, 0)),
          pl.BlockSpec((RPS, 16), lambda i: (i, 0)),
      ],
      out_specs=pl.BlockSpec((RPS, 16), lambda i: (i, 0)),
      out_shape=jax.ShapeDtypeStruct((NPAD, 16), jnp.float32),
  )(da, db)


def _pass2_body(hm, ex, invd, srcp, dstp, z16, vparts,
                srcb, dstb, exb, ivb, hsb, msgb, vsh, semA, semB):
  c = lax.axis_index("c")
  s = lax.axis_index("s")
  wid = c * NS + s

  pltpu.sync_copy(z16.at[pl.ds(s * RPS, RPS)], vsh.at[pl.ds(s * RPS, RPS)])
  plsc.subcore_barrier()

  ebase = wid * E_PER_W

  def issueA(cc, b):
    base = ebase + cc * CHUNK
    pltpu.async_copy(srcp.at[pl.ds(base, CHUNK)], srcb.at[b], semA.at[b])
    pltpu.async_copy(dstp.at[pl.ds(base, CHUNK)], dstb.at[b], semA.at[b])
    pltpu.async_copy(ex.at[pl.ds(base, CHUNK)], exb.at[b], semA.at[b])

  def waitA(b):
    pltpu.make_async_copy(srcp.at[pl.ds(0, CHUNK)], srcb.at[b], semA.at[b]).wait()
    pltpu.make_async_copy(dstp.at[pl.ds(0, CHUNK)], dstb.at[b], semA.at[b]).wait()
    pltpu.make_async_copy(ex.at[pl.ds(0, CHUNK)], exb.at[b], semA.at[b]).wait()

  def issueB(b):
    pltpu.async_copy(invd.at[dstb.at[b]], ivb.at[b], semB.at[b])
    pltpu.async_copy(hm.at[srcb.at[b]], hsb.at[b], semB.at[b])

  def waitB(b):
    pltpu.make_async_copy(invd.at[dstb.at[b]], ivb.at[b], semB.at[b]).wait()
    pltpu.make_async_copy(hm.at[srcb.at[b]], hsb.at[b], semB.at[b]).wait()

  issueA(0, 0)
  issueA(1, 1)
  waitA(0)
  issueB(0)

  def step(i, carry):
    for b in range(2):
      cc = 2 * i + b
      b2 = 1 - b

      @pl.when(cc + 1 < NCHUNK)
      def _():
        waitA(b2)
        issueB(b2)
      waitB(b)
      for j in range(CHUNK // 16):
        rows = _col(j * 16) + _iota16()
        a0 = plsc.load_gather(exb.at[b], [rows, _col(0)]) * plsc.load_gather(
            ivb.at[b], [rows, _col(0)]) * 0.25
        a1 = plsc.load_gather(exb.at[b], [rows, _col(1)]) * plsc.load_gather(
            ivb.at[b], [rows, _col(1)]) * 0.25
        a2 = plsc.load_gather(exb.at[b], [rows, _col(2)]) * plsc.load_gather(
            ivb.at[b], [rows, _col(2)]) * 0.25
        a3 = plsc.load_gather(exb.at[b], [rows, _col(3)]) * plsc.load_gather(
            ivb.at[b], [rows, _col(3)]) * 0.25
        for f in range(F):
          m = a0 * plsc.load_gather(hsb.at[b], [rows, _col(f)])
          m = m + a1 * plsc.load_gather(hsb.at[b], [rows, _col(16 + f)])
          m = m + a2 * plsc.load_gather(hsb.at[b], [rows, _col(32 + f)])
          m = m + a3 * plsc.load_gather(hsb.at[b], [rows, _col(48 + f)])
          plsc.store_scatter(msgb.at[b], [rows, _col(f)], m)
      pltpu.sync_copy(msgb.at[b], vsh.at[dstb.at[b]], add=True)

      @pl.when(cc + 2 < NCHUNK)
      def _():
        issueA(cc + 2, b)
    return carry

  lax.fori_loop(0, NCHUNK // 2, step, 0)
  plsc.subcore_barrier()
  pltpu.sync_copy(vsh.at[pl.ds(s * RPS, RPS)],
                  vparts.at[c, pl.ds(s * RPS, RPS)])


def _pass2(hm, ex, invd, srcp, dstp, z16):
  return pl.kernel(
      _pass2_body,
      out_type=jax.ShapeDtypeStruct((NC, NPAD, 16), jnp.float32),
      mesh=_mesh,
      scratch_types=[
          pltpu.VMEM((2, CHUNK), jnp.int32),
          pltpu.VMEM((2, CHUNK), jnp.int32),
          pltpu.VMEM((2, CHUNK, 4), jnp.float32),
          pltpu.VMEM((2, CHUNK, 16), jnp.float32),
          pltpu.VMEM((2, CHUNK, HF), jnp.float32),
          pltpu.VMEM((2, CHUNK, 16), jnp.float32),
          pltpu.VMEM_SHARED((NPAD, 16), jnp.float32),
          pltpu.SemaphoreType.DMA((2,)),
          pltpu.SemaphoreType.DMA((2,)),
      ],
      compiler_params=pltpu.CompilerParams(needs_layout_passes=False, use_tc_tiling_on_sc=False),
  )(hm, ex, invd, srcp, dstp, z16)


def _comb_body(a_ref, b_ref, o_ref):
  o_ref[...] = a_ref[...] + b_ref[...]


def _combine(va, vb):
  return pl.pallas_call(
      _comb_body,
      grid=(N // BN,),
      in_specs=[
          pl.BlockSpec((BN, 16), lambda i: (i, 0)),
          pl.BlockSpec((BN, 16), lambda i: (i, 0)),
      ],
      out_specs=pl.BlockSpec((BN, 16), lambda i: (i, 0)),
      out_shape=jax.ShapeDtypeStruct((N, 16), jnp.float32),
  )(va, vb)


@jax.jit
def kernel(x, edge_index, W_v, b_v, W_q, b_q, W_k, b_k):
  wqk = jnp.concatenate(
      [W_q, W_k, jnp.zeros((HF, 16 - 2 * H), jnp.float32)], axis=1)
  bqk2 = jnp.concatenate(
      [b_q, b_k, jnp.zeros((16 - 2 * H,), jnp.float32)]).reshape(1, 16)
  bv2 = b_v.reshape(1, HF)

  hm, qk, qkmax = _proj(x, W_v, bv2, wqk, bqk2)
  qm = qkmax[0]
  shift4 = jnp.maximum(qm[:H] + qm[H:2 * H], 0.0)
  shift = jnp.concatenate([shift4, jnp.zeros((12,), jnp.float32)])

  pad = jnp.zeros((EPAD - E,), jnp.int32)
  srcp = jnp.concatenate([edge_index[0], pad])
  dstp = jnp.concatenate([edge_index[1], pad])

  z16 = jnp.zeros((NPAD, 16), jnp.float32)

  ex, dparts = _pass1(qk, srcp, dstp, shift, z16)
  invd = _invd(dparts[0], dparts[1])
  vparts = _pass2(hm, ex, invd, srcp, dstp, z16)
  return _combine(vparts[0], vparts[1])


# speedup vs baseline: 63.2187x; 1.1520x over previous
import functools

import jax
import jax.numpy as jnp
from jax import lax
from jax.experimental import pallas as pl
from jax.experimental.pallas import tpu as pltpu
from jax.experimental.pallas import tpu_sc as plsc

N = 50000
E = 800000
IN = 128
H = 4
F = 16
HF = H * F

NC = 2
NS = 16
NW = NC * NS

CHUNK = 128
E_PER_W = 25600
EPAD = E_PER_W * NW
NCHUNK = E_PER_W // CHUNK
RPS = 3128
NPAD = RPS * NS

_mesh = plsc.VectorSubcoreMesh(core_axis_name="c", subcore_axis_name="s")


def _iota16():
  return lax.iota(jnp.int32, 16)


def _col(v):
  return jnp.full((16,), v, jnp.int32)


BN = 2000
BNP = 6256


def _proj_body(x_ref, wv_ref, bv_ref, wqk_ref, bqk_ref, h_ref, hb_ref,
               qk_ref, qkmax_ref):
  xb = x_ref[...]
  h = jnp.dot(xb, wv_ref[...], preferred_element_type=jnp.float32) + bv_ref[...]
  qk = jnp.dot(h, wqk_ref[...], preferred_element_type=jnp.float32) + bqk_ref[...]
  h_ref[...] = h
  hb_ref[...] = h.astype(jnp.bfloat16)
  qk_ref[...] = qk
  rows = jax.lax.broadcasted_iota(jnp.int32, (BNP, 16), 0) + pl.program_id(0) * BNP
  bm = jnp.max(jnp.where(rows < N, qk, -jnp.inf), axis=0, keepdims=True)

  @pl.when(pl.program_id(0) == 0)
  def _():
    qkmax_ref[...] = bm

  @pl.when(pl.program_id(0) > 0)
  def _():
    qkmax_ref[...] = jnp.maximum(qkmax_ref[...], bm)


def _proj(x, wv, bv2, wqk, bqk2):
  return pl.pallas_call(
      _proj_body,
      grid=(NPAD // BNP,),
      in_specs=[
          pl.BlockSpec((BNP, IN), lambda i: (i, 0)),
          pl.BlockSpec((IN, HF), lambda i: (0, 0)),
          pl.BlockSpec((1, HF), lambda i: (0, 0)),
          pl.BlockSpec((HF, 16), lambda i: (0, 0)),
          pl.BlockSpec((1, 16), lambda i: (0, 0)),
      ],
      out_specs=[
          pl.BlockSpec((BNP, HF), lambda i: (i, 0)),
          pl.BlockSpec((BNP, HF), lambda i: (i, 0)),
          pl.BlockSpec((BNP, 16), lambda i: (i, 0)),
          pl.BlockSpec((1, 16), lambda i: (0, 0)),
      ],
      out_shape=[
          jax.ShapeDtypeStruct((NPAD, HF), jnp.float32),
          jax.ShapeDtypeStruct((NPAD, HF), jnp.bfloat16),
          jax.ShapeDtypeStruct((NPAD, 16), jnp.float32),
          jax.ShapeDtypeStruct((1, 16), jnp.float32),
      ],
  )(x, wv, bv2, wqk, bqk2)


def _pass1_body(qk, srcp, dstp, shift, z16, ex_out, dparts,
                shiftb, srcb, dstb, qs, kd, exb, exb16, denom_sh,
                semA, semB, semW):
  c = lax.axis_index("c")
  s = lax.axis_index("s")
  wid = c * NS + s

  pltpu.sync_copy(shift.at[pl.ds(0, 16)], shiftb)
  pltpu.sync_copy(z16.at[pl.ds(s * RPS, RPS)], denom_sh.at[pl.ds(s * RPS, RPS)])
  for r in range(CHUNK):
    exb16[0, r, :] = jnp.zeros((16,), jnp.float32)
    exb16[1, r, :] = jnp.zeros((16,), jnp.float32)
  plsc.subcore_barrier()

  shv = shiftb[...]
  shs = (shv[0], shv[1], shv[2], shv[3])
  ebase = wid * E_PER_W

  def issueA(cc, b):
    base = ebase + cc * CHUNK
    pltpu.async_copy(srcp.at[pl.ds(base, CHUNK)], srcb.at[b], semA.at[b])
    pltpu.async_copy(dstp.at[pl.ds(base, CHUNK)], dstb.at[b], semA.at[b])

  def waitA(b):
    pltpu.make_async_copy(srcp.at[pl.ds(0, CHUNK)], srcb.at[b], semA.at[b]).wait()
    pltpu.make_async_copy(dstp.at[pl.ds(0, CHUNK)], dstb.at[b], semA.at[b]).wait()

  def issueB(b):
    pltpu.async_copy(qk.at[srcb.at[b]], qs.at[b], semB.at[b])
    pltpu.async_copy(qk.at[dstb.at[b]], kd.at[b], semB.at[b])

  def waitB(b):
    pltpu.make_async_copy(qk.at[srcb.at[b]], qs.at[b], semB.at[b]).wait()
    pltpu.make_async_copy(qk.at[dstb.at[b]], kd.at[b], semB.at[b]).wait()

  issueA(0, 0)
  issueA(1, 1)
  waitA(0)
  issueB(0)

  def step(i, carry):
    for b in range(2):
      cc = 2 * i + b
      b2 = 1 - b
      @pl.when(cc + 1 < NCHUNK)
      def _():
        waitA(b2)
        issueB(b2)
      waitB(b)
      base = ebase + cc * CHUNK

      @pl.when(cc >= 2)
      def _():
        pltpu.make_async_copy(exb.at[b], ex_out.at[pl.ds(0, CHUNK)],
                              semW.at[b]).wait()
      for j in range(CHUNK // 16):
        rows = _col(j * 16) + _iota16()
        gid = base + j * 16 + _iota16()
        valid = gid < E
        for h in range(4):
          qv = plsc.load_gather(qs.at[b], [rows, _col(h)])
          kv = plsc.load_gather(kd.at[b], [rows, _col(4 + h)])
          e = qv + kv
          cf = jnp.where(e >= 0.0, e, 0.2 * e)
          ex = jnp.exp(cf - shs[h])
          ex = jnp.where(valid, ex, 0.0)
          plsc.store_scatter(exb.at[b], [rows, _col(h)], ex)
          plsc.store_scatter(exb16.at[b], [rows, _col(h)], ex)
      pltpu.async_copy(exb.at[b], ex_out.at[pl.ds(base, CHUNK)], semW.at[b])
      pltpu.sync_copy(exb16.at[b], denom_sh.at[dstb.at[b]], add=True)

      @pl.when(cc + 2 < NCHUNK)
      def _():
        issueA(cc + 2, b)
    return carry

  lax.fori_loop(0, NCHUNK // 2, step, 0)
  for b in range(2):
    pltpu.make_async_copy(exb.at[b], ex_out.at[pl.ds(0, CHUNK)],
                          semW.at[b]).wait()
  plsc.subcore_barrier()
  pltpu.sync_copy(denom_sh.at[pl.ds(s * RPS, RPS)],
                  dparts.at[c, pl.ds(s * RPS, RPS)])


def _pass1(qk, srcp, dstp, shift, z16):
  return pl.kernel(
      _pass1_body,
      out_type=[
          jax.ShapeDtypeStruct((EPAD, 4), jnp.float32),
          jax.ShapeDtypeStruct((NC, NPAD, 16), jnp.float32),
      ],
      mesh=_mesh,
      scratch_types=[
          pltpu.VMEM((16,), jnp.float32),
          pltpu.VMEM((2, CHUNK), jnp.int32),
          pltpu.VMEM((2, CHUNK), jnp.int32),
          pltpu.VMEM((2, CHUNK, 16), jnp.float32),
          pltpu.VMEM((2, CHUNK, 16), jnp.float32),
          pltpu.VMEM((2, CHUNK, 4), jnp.float32),
          pltpu.VMEM((2, CHUNK, 16), jnp.float32),
          pltpu.VMEM_SHARED((NPAD, 16), jnp.float32),
          pltpu.SemaphoreType.DMA((2,)),
          pltpu.SemaphoreType.DMA((2,)),
          pltpu.SemaphoreType.DMA((2,)),
      ],
      compiler_params=pltpu.CompilerParams(needs_layout_passes=False, use_tc_tiling_on_sc=False),
  )(qk, srcp, dstp, shift, z16)


def _invd_body(a_ref, b_ref, o_ref):
  o_ref[...] = 1.0 / (a_ref[...] + b_ref[...] + 1e-9)


def _invd(da, db):
  return pl.pallas_call(
      _invd_body,
      grid=(NS,),
      in_specs=[
          pl.BlockSpec((RPS, 16), lambda i: (i, 0)),
          pl.BlockSpec((RPS, 16), lambda i: (i, 0)),
      ],
      out_specs=pl.BlockSpec((RPS, 16), lambda i: (i, 0)),
      out_shape=jax.ShapeDtypeStruct((NPAD, 16), jnp.float32),
  )(da, db)


def _pass2_body(hm, ex, invd, srcp, dstp, z16, vparts,
                srcb, dstb, exb, ivb, hsb, msgb, vsh, semA, semB):
  c = lax.axis_index("c")
  s = lax.axis_index("s")
  wid = c * NS + s

  pltpu.sync_copy(z16.at[pl.ds(s * RPS, RPS)], vsh.at[pl.ds(s * RPS, RPS)])
  plsc.subcore_barrier()

  ebase = wid * E_PER_W

  def issueA(cc, b):
    base = ebase + cc * CHUNK
    pltpu.async_copy(srcp.at[pl.ds(base, CHUNK)], srcb.at[b], semA.at[b])
    pltpu.async_copy(dstp.at[pl.ds(base, CHUNK)], dstb.at[b], semA.at[b])
    pltpu.async_copy(ex.at[pl.ds(base, CHUNK)], exb.at[b], semA.at[b])

  def waitA(b):
    pltpu.make_async_copy(srcp.at[pl.ds(0, CHUNK)], srcb.at[b], semA.at[b]).wait()
    pltpu.make_async_copy(dstp.at[pl.ds(0, CHUNK)], dstb.at[b], semA.at[b]).wait()
    pltpu.make_async_copy(ex.at[pl.ds(0, CHUNK)], exb.at[b], semA.at[b]).wait()

  def issueB(b):
    pltpu.async_copy(invd.at[dstb.at[b]], ivb.at[b], semB.at[b])
    pltpu.async_copy(hm.at[srcb.at[b]], hsb.at[b], semB.at[b])

  def waitB(b):
    pltpu.make_async_copy(invd.at[dstb.at[b]], ivb.at[b], semB.at[b]).wait()
    pltpu.make_async_copy(hm.at[srcb.at[b]], hsb.at[b], semB.at[b]).wait()

  issueA(0, 0)
  issueA(1, 1)
  waitA(0)
  issueB(0)

  def step(i, carry):
    for b in range(2):
      cc = 2 * i + b
      b2 = 1 - b

      @pl.when(cc + 1 < NCHUNK)
      def _():
        waitA(b2)
        issueB(b2)
      waitB(b)
      himask = jnp.full((16,), -65536, jnp.int32)
      for j in range(CHUNK // 16):
        rows = _col(j * 16) + _iota16()
        aa = []
        for h in range(4):
          aa.append(
              plsc.load_gather(exb.at[b], [rows, _col(h)]) *
              plsc.load_gather(ivb.at[b], [rows, _col(h)]) * 0.25)
        for k in range(8):
          me = jnp.zeros((16,), jnp.float32)
          mo = jnp.zeros((16,), jnp.float32)
          for h in range(4):
            w = plsc.load_gather(hsb.at[b], [rows, _col(h * 8 + k)])
            lo = jax.lax.bitcast_convert_type(
                jax.lax.shift_left(w, 16), jnp.float32)
            hi = jax.lax.bitcast_convert_type(w & himask, jnp.float32)
            me = me + aa[h] * lo
            mo = mo + aa[h] * hi
          plsc.store_scatter(msgb.at[b], [rows, _col(2 * k)], me)
          plsc.store_scatter(msgb.at[b], [rows, _col(2 * k + 1)], mo)
      pltpu.sync_copy(msgb.at[b], vsh.at[dstb.at[b]], add=True)

      @pl.when(cc + 2 < NCHUNK)
      def _():
        issueA(cc + 2, b)
    return carry

  lax.fori_loop(0, NCHUNK // 2, step, 0)
  plsc.subcore_barrier()
  pltpu.sync_copy(vsh.at[pl.ds(s * RPS, RPS)],
                  vparts.at[c, pl.ds(s * RPS, RPS)])


def _pass2(hm, ex, invd, srcp, dstp, z16):
  return pl.kernel(
      _pass2_body,
      out_type=jax.ShapeDtypeStruct((NC, NPAD, 16), jnp.float32),
      mesh=_mesh,
      scratch_types=[
          pltpu.VMEM((2, CHUNK), jnp.int32),
          pltpu.VMEM((2, CHUNK), jnp.int32),
          pltpu.VMEM((2, CHUNK, 4), jnp.float32),
          pltpu.VMEM((2, CHUNK, 16), jnp.float32),
          pltpu.VMEM((2, CHUNK, 32), jnp.int32),
          pltpu.VMEM((2, CHUNK, 16), jnp.float32),
          pltpu.VMEM_SHARED((NPAD, 16), jnp.float32),
          pltpu.SemaphoreType.DMA((2,)),
          pltpu.SemaphoreType.DMA((2,)),
      ],
      compiler_params=pltpu.CompilerParams(needs_layout_passes=False, use_tc_tiling_on_sc=False),
  )(hm, ex, invd, srcp, dstp, z16)


def _comb_body(a_ref, b_ref, o_ref):
  o_ref[...] = a_ref[...] + b_ref[...]


def _combine(va, vb):
  return pl.pallas_call(
      _comb_body,
      grid=(N // BN,),
      in_specs=[
          pl.BlockSpec((BN, 16), lambda i: (i, 0)),
          pl.BlockSpec((BN, 16), lambda i: (i, 0)),
      ],
      out_specs=pl.BlockSpec((BN, 16), lambda i: (i, 0)),
      out_shape=jax.ShapeDtypeStruct((N, 16), jnp.float32),
  )(va, vb)


@jax.jit
def kernel(x, edge_index, W_v, b_v, W_q, b_q, W_k, b_k):
  wqk = jnp.concatenate(
      [W_q, W_k, jnp.zeros((HF, 16 - 2 * H), jnp.float32)], axis=1)
  bqk2 = jnp.concatenate(
      [b_q, b_k, jnp.zeros((16 - 2 * H,), jnp.float32)]).reshape(1, 16)
  bv2 = b_v.reshape(1, HF)

  hm, hb, qk, qkmax = _proj(x, W_v, bv2, wqk, bqk2)
  hm32 = jax.lax.bitcast_convert_type(hb.reshape(NPAD, 32, 2), jnp.int32)
  qm = qkmax[0]
  shift4 = jnp.maximum(qm[:H] + qm[H:2 * H], 0.0)
  shift = jnp.concatenate([shift4, jnp.zeros((12,), jnp.float32)])

  pad = jnp.zeros((EPAD - E,), jnp.int32)
  srcp = jnp.concatenate([edge_index[0], pad])
  dstp = jnp.concatenate([edge_index[1], pad])

  z16 = jnp.zeros((NPAD, 16), jnp.float32)

  ex, dparts = _pass1(qk, srcp, dstp, shift, z16)
  invd = _invd(dparts[0], dparts[1])
  vparts = _pass2(hm32, ex, invd, srcp, dstp, z16)
  return _combine(vparts[0], vparts[1])
